# Initial kernel scaffold; baseline (speedup 1.0000x reference)
import functools

import jax
import jax.numpy as jnp
from jax import lax
from jax.experimental import pallas as pl
from jax.experimental.pallas import tpu as pltpu
from jax.experimental.pallas import tpu_sc as plsc

_BATCH = 4096
_HIST = 200
_DIM = 64
_N = _BATCH * _HIST
_NW = 32
_PER_W = _N // _NW
_CHUNK = 128
_NCHUNK = _PER_W // _CHUNK


def _make_kernel():
    mesh = plsc.VectorSubcoreMesh(core_axis_name="c", subcore_axis_name="s")

    @functools.partial(
        pl.kernel,
        mesh=mesh,
        out_type=(
            jax.ShapeDtypeStruct((_N, _DIM), jnp.float32),
            jax.ShapeDtypeStruct((_N, _DIM), jnp.float32),
        ),
        scratch_types=[
            pltpu.VMEM((_NCHUNK, _CHUNK), jnp.int32),
            pltpu.VMEM((_CHUNK, _DIM), jnp.float32),
            pltpu.VMEM((_CHUNK, _DIM), jnp.float32),
            pltpu.SemaphoreType.DMA,
            pltpu.SemaphoreType.DMA,
        ],
    )
    def dual_gather(idx_hbm, st_hbm, nst_hbm, out_st, out_nst,
                    idx_v, rows_st, rows_nst, sem_st, sem_nst):
        wid = lax.axis_index("s") * 2 + lax.axis_index("c")
        pltpu.sync_copy(idx_hbm.at[wid], idx_v)

        def step(j, carry):
            base = wid * _PER_W + j * _CHUNK
            cp_st = pltpu.async_copy(st_hbm.at[idx_v.at[j]], rows_st, sem_st)
            cp_nst = pltpu.async_copy(nst_hbm.at[idx_v.at[j]], rows_nst, sem_nst)
            cp_st.wait()
            pltpu.sync_copy(rows_st, out_st.at[pl.ds(base, _CHUNK)])
            cp_nst.wait()
            pltpu.sync_copy(rows_nst, out_nst.at[pl.ds(base, _CHUNK)])
            return carry

        lax.fori_loop(0, _NCHUNK, step, 0)

    return dual_gather


_DUAL_GATHER = _make_kernel()


@jax.jit
def kernel(inputs, static_table, non_static_table):
    idx = inputs.reshape(_NW, _NCHUNK, _CHUNK)
    out_st, out_nst = _DUAL_GATHER(idx, static_table, non_static_table)
    return (out_st.reshape(_BATCH, _HIST, _DIM),
            out_nst.reshape(_BATCH, _HIST, _DIM))


# speedup vs baseline: 4.6267x; 4.6267x over previous
import functools

import jax
import jax.numpy as jnp
from jax import lax
from jax.experimental import pallas as pl
from jax.experimental.pallas import tpu as pltpu
from jax.experimental.pallas import tpu_sc as plsc

_BATCH = 4096
_HIST = 200
_DIM = 64
_N = _BATCH * _HIST
_NW = 32
_PER_W = _N // _NW
_CHUNK = 128
_NCHUNK = _PER_W // _CHUNK


def _make_kernel():
    mesh = plsc.VectorSubcoreMesh(core_axis_name="c", subcore_axis_name="s")

    @functools.partial(
        pl.kernel,
        mesh=mesh,
        out_type=jax.ShapeDtypeStruct((_N, 2 * _DIM), jnp.float32),
        scratch_types=[
            pltpu.VMEM((_NCHUNK, _CHUNK), jnp.int32),
            pltpu.VMEM((_CHUNK, 2 * _DIM), jnp.float32),
            pltpu.VMEM((_CHUNK, 2 * _DIM), jnp.float32),
            pltpu.SemaphoreType.DMA,
            pltpu.SemaphoreType.DMA,
        ],
    )
    def dual_gather(idx_hbm, tab_hbm, out_hbm, idx_v, rows_a, rows_b, sem_a, sem_b):
        wid = lax.axis_index("s") * 2 + lax.axis_index("c")
        pltpu.sync_copy(idx_hbm.at[wid], idx_v)
        base = wid * _PER_W

        pltpu.async_copy(tab_hbm.at[idx_v.at[0]], rows_a, sem_a)

        def step(i, carry):
            j = 2 * i
            pltpu.async_copy(tab_hbm.at[idx_v.at[j + 1]], rows_b, sem_b)
            pltpu.make_async_copy(tab_hbm.at[idx_v.at[j]], rows_a, sem_a).wait()
            pltpu.sync_copy(rows_a, out_hbm.at[pl.ds(base + j * _CHUNK, _CHUNK)])

            @pl.when(j + 2 < _NCHUNK)
            def _():
                pltpu.async_copy(tab_hbm.at[idx_v.at[j + 2]], rows_a, sem_a)

            pltpu.make_async_copy(tab_hbm.at[idx_v.at[j + 1]], rows_b, sem_b).wait()
            pltpu.sync_copy(rows_b, out_hbm.at[pl.ds(base + (j + 1) * _CHUNK, _CHUNK)])
            return carry

        lax.fori_loop(0, _NCHUNK // 2, step, 0)

    return dual_gather


_DUAL_GATHER = _make_kernel()


@jax.jit
def kernel(inputs, static_table, non_static_table):
    idx = inputs.reshape(_NW, _NCHUNK, _CHUNK)
    fused = jnp.concatenate([static_table, non_static_table], axis=1)
    out = _DUAL_GATHER(idx, fused)
    return (out[:, :_DIM].reshape(_BATCH, _HIST, _DIM),
            out[:, _DIM:].reshape(_BATCH, _HIST, _DIM))
